# Initial kernel scaffold; baseline (speedup 1.0000x reference)
import jax
import jax.numpy as jnp
from jax.experimental import pallas as pl


def kernel(features_1, edge_index_1, target, W_e1, b_e1, W_e2, b_e2, W_e3, b_e3, W_c1, b_c1, W_c2, b_c2, W_att, W_fc, b_fc):
    raise NotImplementedError("write your pallas kernel here")


# speedup vs baseline: 2.4157x; 2.4157x over previous
import jax
import jax.numpy as jnp
from jax.experimental import pallas as pl

N = 15
E = 210
EL = E + N


def _dot(a, b):
    return jax.lax.dot_general(a, b, (((1,), (0,)), ((), ())),
                               precision=jax.lax.Precision.HIGHEST,
                               preferred_element_type=jnp.float32)


def _dot_t(a, b):
    return jax.lax.dot_general(a, b, (((1,), (1,)), ((), ())),
                               precision=jax.lax.Precision.HIGHEST,
                               preferred_element_type=jnp.float32)


def _body(f_ref, s_ref, d_ref, tgt_ref,
          we1_ref, be1_ref, we2_ref, be2_ref, we3_ref, be3_ref,
          wc1_ref, bc1_ref, wc2_ref, bc2_ref,
          watt_ref, wfc_ref, bfc_ref,
          loss_ref, preds_ref):
    x0 = jnp.maximum(_dot_t(f_ref[...], we1_ref[...]) + be1_ref[...], 0.0)
    x1 = jnp.maximum(_dot_t(x0, we2_ref[...]) + be2_ref[...], 0.0)
    x2f = _dot_t(x1, we3_ref[...]) + be3_ref[...]
    x2 = jnp.concatenate([x2f[:, 128 * n:128 * (n + 1)] for n in range(N)],
                         axis=0)

    s_ids = s_ref[...]
    d_ids = d_ref[...]
    nodes = jax.lax.broadcasted_iota(jnp.int32, (N, EL), 0)
    s_oh = (nodes == s_ids).astype(jnp.float32)
    d_oh = (nodes == d_ids).astype(jnp.float32)
    adj = _dot_t(d_oh, s_oh)
    ones_row = jnp.ones((1, N), jnp.float32)
    ones_col = jnp.ones((N, 1), jnp.float32)
    deg_col = _dot(adj, ones_col)
    deg_row = _dot_t(ones_row, adj)
    dis_col = jnp.where(deg_col > 0, jax.lax.rsqrt(deg_col), 0.0)
    dis_row = jnp.where(deg_row > 0, jax.lax.rsqrt(deg_row), 0.0)
    a_norm = adj * dis_col * dis_row

    h1 = _dot(a_norm, _dot_t(x2, wc1_ref[...])) + bc1_ref[...]
    h1 = jnp.maximum(h1, 0.0)
    h2 = _dot(a_norm, _dot_t(h1, wc2_ref[...])) + bc2_ref[...]

    gc = _dot(ones_row, _dot(h2, watt_ref[...])) * (1.0 / N)
    tg = jnp.tanh(gc)
    scores = jax.nn.sigmoid(_dot_t(h2, tg))
    rep = jnp.sum(h2 * scores, axis=0, keepdims=True)
    logits = _dot_t(rep, wfc_ref[...]) + bfc_ref[...]

    tgt = tgt_ref[...]
    idx3 = jax.lax.broadcasted_iota(jnp.int32, (1, 3), 1)
    tmax = jnp.max(tgt, axis=1, keepdims=True)
    label = jnp.min(jnp.where(tgt >= tmax, idx3, 3), axis=1, keepdims=True)
    m = jnp.max(logits, axis=1, keepdims=True)
    ex = jnp.exp(logits - m)
    sex = jnp.sum(ex, axis=1, keepdims=True)
    logsm = logits - m - jnp.log(sex)
    loss_ref[...] = -jnp.sum(jnp.where(idx3 == label, logsm, 0.0),
                             axis=1, keepdims=True)
    preds_ref[...] = ex / sex


def kernel(features_1, edge_index_1, target, W_e1, b_e1, W_e2, b_e2,
           W_e3, b_e3, W_c1, b_c1, W_c2, b_c2, W_att, W_fc, b_fc):
    loop = jnp.arange(N, dtype=edge_index_1.dtype)
    s = jnp.concatenate([edge_index_1[0], loop]).reshape(1, EL)
    d = jnp.concatenate([edge_index_1[1], loop]).reshape(1, EL)
    f = features_1.reshape(1, N)
    args = (f, s, d, target.reshape(1, 3),
            W_e1, b_e1.reshape(1, -1), W_e2, b_e2.reshape(1, -1),
            W_e3, b_e3.reshape(1, -1),
            W_c1, b_c1.reshape(1, -1), W_c2, b_c2.reshape(1, -1),
            W_att, W_fc, b_fc.reshape(1, -1))
    loss2d, preds2d = pl.pallas_call(
        _body,
        out_shape=(jax.ShapeDtypeStruct((1, 1), jnp.float32),
                   jax.ShapeDtypeStruct((1, 3), jnp.float32)),
    )(*args)
    return (loss2d[0, 0], preds2d[0])
